# baseline (speedup 1.0000x reference)
import functools

import jax
import jax.numpy as jnp
from jax import lax
from jax.experimental import pallas as pl
from jax.experimental.pallas import tpu as pltpu
from jax.experimental.pallas import tpu_sc as plsc

VOCAB = 1300
D = 128
BATCH = 16384
HIST = 200
B = BATCH * HIST

NC, NS = 2, 16
NW = NC * NS
C = 256
NCHUNK = B // (NW * C)
SUP = 16
NSUP = NCHUNK // SUP

_MESH = plsc.VectorSubcoreMesh(core_axis_name="c", subcore_axis_name="s")


@functools.partial(
    pl.kernel,
    out_type=jax.ShapeDtypeStruct((B, D), jnp.float32),
    mesh=_MESH,
    scratch_types=[
        pltpu.VMEM((SUP * C,), jnp.int32),
        pltpu.VMEM((C, D), jnp.float32),
        pltpu.VMEM((C, D), jnp.float32),
        pltpu.VMEM_SHARED((VOCAB, D), jnp.float32),
        pltpu.SemaphoreType.DMA,
        pltpu.SemaphoreType.DMA,
        pltpu.SemaphoreType.DMA,
        pltpu.SemaphoreType.DMA,
    ],
)
def _sc_gather(
    idx_hbm,
    table_hbm,
    out_hbm,
    idx_v,
    rows0,
    rows1,
    table_sh,
    gsem0,
    gsem1,
    ssem0,
    ssem1,
):
    wid = lax.axis_index("s") * NC + lax.axis_index("c")
    chunk0 = wid * NCHUNK

    @pl.when(lax.axis_index("s") == 0)
    def _():
        pltpu.sync_copy(table_hbm, table_sh)

    plsc.subcore_barrier()

    def start_gather(j, rows_v, gsem):
        pltpu.async_copy(table_sh.at[idx_v.at[pl.ds(j * C, C)]], rows_v, gsem)

    def start_gather_hbm(j, rows_v, gsem):
        pltpu.async_copy(table_hbm.at[idx_v.at[pl.ds(j * C, C)]], rows_v, gsem)

    def wait_gather(rows_v, gsem):
        pltpu.make_async_copy(
            table_sh.at[idx_v.at[pl.ds(0, C)]], rows_v, gsem
        ).wait()

    def start_store(row, rows_v, ssem):
        pltpu.async_copy(rows_v, out_hbm.at[pl.ds(row, C)], ssem)

    def wait_store(rows_v, ssem):
        pltpu.make_async_copy(rows_v, out_hbm.at[pl.ds(0, C)], ssem).wait()

    def superchunk(s, _):
        sup_row = (chunk0 + s * SUP) * C

        pltpu.sync_copy(idx_hbm.at[pl.ds(sup_row, SUP * C)], idx_v)

        @pl.when(s > 0)
        def _():
            wait_store(rows0, ssem0)

        start_gather(0, rows0, gsem0)

        def pair(g, _):
            @pl.when((s > 0) | (g > 0))
            def _():
                wait_store(rows1, ssem1)

            @pl.when(g % 2 == 0)
            def _():
                start_gather_hbm(2 * g + 1, rows1, gsem1)

            @pl.when(g % 2 == 1)
            def _():
                start_gather(2 * g + 1, rows1, gsem1)

            wait_gather(rows0, gsem0)
            start_store(sup_row + 2 * g * C, rows0, ssem0)

            @pl.when(g < SUP // 2 - 1)
            def _():
                wait_store(rows0, ssem0)
                start_gather(2 * g + 2, rows0, gsem0)

            wait_gather(rows1, gsem1)
            start_store(sup_row + (2 * g + 1) * C, rows1, ssem1)
            return 0

        lax.fori_loop(0, SUP // 2, pair, 0)
        return 0

    lax.fori_loop(0, NSUP, superchunk, 0)
    wait_store(rows0, ssem0)
    wait_store(rows1, ssem1)


def kernel(indices, species):
    idx = indices.reshape(-1).astype(jnp.int32)
    out = _sc_gather(idx, species)
    return out.reshape(BATCH, HIST, D)


# speedup vs baseline: 1.2114x; 1.2114x over previous
import functools

import jax
import jax.numpy as jnp
from jax import lax
from jax.experimental import pallas as pl
from jax.experimental.pallas import tpu as pltpu
from jax.experimental.pallas import tpu_sc as plsc

VOCAB = 1300
D = 128
BATCH = 16384
HIST = 200
B = BATCH * HIST

NC, NS = 2, 16
NW = NC * NS
C = 128
NBUF = 4
NCHUNK = B // (NW * C)
SUP = 32
NSUP = NCHUNK // SUP

_MESH = plsc.VectorSubcoreMesh(core_axis_name="c", subcore_axis_name="s")


@functools.partial(
    pl.kernel,
    out_type=jax.ShapeDtypeStruct((B, D), jnp.float32),
    mesh=_MESH,
    scratch_types=[
        pltpu.VMEM((SUP * C,), jnp.int32),
        [pltpu.VMEM((C, D), jnp.float32) for _ in range(NBUF)],
        pltpu.VMEM_SHARED((VOCAB, D), jnp.float32),
        [pltpu.SemaphoreType.DMA for _ in range(NBUF)],
        [pltpu.SemaphoreType.DMA for _ in range(NBUF)],
    ],
)
def _sc_gather(idx_hbm, table_hbm, out_hbm, idx_v, rows, table_sh, gsems, ssems):
    wid = lax.axis_index("s") * NC + lax.axis_index("c")
    chunk0 = wid * NCHUNK

    @pl.when(lax.axis_index("s") == 0)
    def _():
        pltpu.sync_copy(table_hbm, table_sh)

    plsc.subcore_barrier()

    def start_gather(j, b):
        pltpu.async_copy(table_sh.at[idx_v.at[pl.ds(j * C, C)]], rows[b], gsems[b])

    def wait_gather(b):
        pltpu.make_async_copy(
            table_sh.at[idx_v.at[pl.ds(0, C)]], rows[b], gsems[b]
        ).wait()

    def start_store(row, b):
        pltpu.async_copy(rows[b], out_hbm.at[pl.ds(row, C)], ssems[b])

    def wait_store(b):
        pltpu.make_async_copy(rows[b], out_hbm.at[pl.ds(0, C)], ssems[b]).wait()

    def superchunk(s, _):
        sup_row = (chunk0 + s * SUP) * C

        pltpu.sync_copy(idx_hbm.at[pl.ds(sup_row, SUP * C)], idx_v)

        @pl.when(s > 0)
        def _():
            wait_store(0)

        start_gather(0, 0)

        def group(g, _):
            for b in range(NBUF):
                j = 4 * g + b
                wait_gather(b)
                start_store(sup_row + j * C, b)
                nb = (b + 1) % NBUF
                if b < NBUF - 1:
                    @pl.when((s > 0) | (g > 0))
                    def _():
                        wait_store(nb)

                    start_gather(j + 1, nb)
                else:
                    @pl.when(g < SUP // NBUF - 1)
                    def _():
                        wait_store(nb)
                        start_gather(j + 1, nb)
            return 0

        lax.fori_loop(0, SUP // NBUF, group, 0)
        return 0

    lax.fori_loop(0, NSUP, superchunk, 0)
    for b in range(NBUF):
        wait_store(b)


def kernel(indices, species):
    idx = indices.reshape(-1).astype(jnp.int32)
    out = _sc_gather(idx, species)
    return out.reshape(BATCH, HIST, D)
